# Initial kernel scaffold; baseline (speedup 1.0000x reference)
import functools

import jax
import jax.numpy as jnp
from jax import lax
from jax.experimental import pallas as pl
from jax.experimental.pallas import tpu as pltpu
from jax.experimental.pallas import tpu_sc as plsc

N = 10000
E = 320000
C = 128
NC = 2
NS = 16
NW = NC * NS
EPW = E // NW
K = 80
NCHUNK = EPW // K
RPS = N // NS
HL = 16

_mesh = plsc.VectorSubcoreMesh(core_axis_name="c", subcore_axis_name="s")


@functools.partial(
    pl.kernel,
    out_type=jax.ShapeDtypeStruct((NC, N, HL), jnp.float32),
    mesh=_mesh,
    scratch_types=[
        pltpu.VMEM((NCHUNK, K), jnp.int32),
        pltpu.VMEM((K, HL), jnp.float32),
        pltpu.VMEM_SHARED((N, HL), jnp.float32),
        pltpu.SemaphoreType.DMA,
    ],
)
def _sc_degree(dst_hbm, ones_hbm, z16_hbm, degp_hbm, idx_v, ones_v, deg_sh, sem):
    c = lax.axis_index("c")
    s = lax.axis_index("s")
    wid = c * NS + s
    pltpu.sync_copy(dst_hbm.at[wid], idx_v)
    pltpu.sync_copy(ones_hbm, ones_v)
    pltpu.sync_copy(z16_hbm.at[pl.ds(s * RPS, RPS)],
                    deg_sh.at[pl.ds(s * RPS, RPS)])
    plsc.subcore_barrier()

    @pl.loop(0, NCHUNK)
    def _(j):
        pltpu.sync_copy(ones_v, deg_sh.at[idx_v.at[j]], add=True)

    plsc.subcore_barrier()
    pltpu.sync_copy(deg_sh.at[pl.ds(s * RPS, RPS)],
                    degp_hbm.at[c, pl.ds(s * RPS, RPS)])


@functools.partial(
    pl.kernel,
    out_type=jax.ShapeDtypeStruct((NC, N, C), jnp.float32),
    mesh=_mesh,
    scratch_types=[
        pltpu.VMEM((NCHUNK, K), jnp.int32),
        pltpu.VMEM((NCHUNK, K), jnp.int32),
        pltpu.VMEM((K, C), jnp.float32),
        pltpu.VMEM((K, C), jnp.float32),
        pltpu.VMEM_SHARED((N, C), jnp.float32),
        pltpu.SemaphoreType.DMA,
        pltpu.SemaphoreType.DMA,
    ],
)
def _sc_aggregate(src_hbm, dst_hbm, h2_hbm, z128_hbm, acc_hbm,
                  si_v, di_v, ra, rb, acc_sh, sa, sb):
    c = lax.axis_index("c")
    s = lax.axis_index("s")
    wid = c * NS + s
    pltpu.sync_copy(src_hbm.at[wid], si_v)
    pltpu.sync_copy(dst_hbm.at[wid], di_v)
    pltpu.sync_copy(z128_hbm.at[pl.ds(s * RPS, RPS)],
                    acc_sh.at[pl.ds(s * RPS, RPS)])
    plsc.subcore_barrier()

    pltpu.make_async_copy(h2_hbm.at[si_v.at[0]], ra, sa).start()

    @pl.loop(0, NCHUNK // 2)
    def _(j):
        c0 = 2 * j
        c1 = c0 + 1
        pltpu.make_async_copy(h2_hbm.at[si_v.at[c1]], rb, sb).start()
        pltpu.make_async_copy(h2_hbm.at[si_v.at[c0]], ra, sa).wait()
        pltpu.sync_copy(ra, acc_sh.at[di_v.at[c0]], add=True)
        pltpu.make_async_copy(h2_hbm.at[si_v.at[c0 + 2]], ra, sa).start()
        pltpu.make_async_copy(h2_hbm.at[si_v.at[c1]], rb, sb).wait()
        pltpu.sync_copy(rb, acc_sh.at[di_v.at[c1]], add=True)

    pltpu.make_async_copy(h2_hbm.at[si_v.at[NCHUNK - 1]], ra, sa).wait()
    pltpu.sync_copy(ra, acc_sh.at[di_v.at[NCHUNK - 1]], add=True)

    plsc.subcore_barrier()
    pltpu.sync_copy(acc_sh.at[pl.ds(s * RPS, RPS)],
                    acc_hbm.at[c, pl.ds(s * RPS, RPS)])


BM = 1000


def _mm_body(x_ref, w_ref, h_ref):
    h_ref[...] = jnp.dot(x_ref[...], w_ref[...],
                         preferred_element_type=jnp.float32)


def _deg_block(degp_ref):
    cnt = jnp.sum(degp_ref[0] + degp_ref[1], axis=-1, keepdims=True)
    return cnt * (1.0 / HL) + 1.0


def _prescale_body(degp_ref, h_ref, h2_ref):
    h2_ref[...] = lax.rsqrt(_deg_block(degp_ref)) * h_ref[...]


def _final_body(degp_ref, acc_ref, h2_ref, b_ref, o_ref):
    dis = lax.rsqrt(_deg_block(degp_ref))
    tot = acc_ref[0] + acc_ref[1] + h2_ref[...]
    o_ref[...] = jnp.maximum(dis * tot + b_ref[...], 0.0)


def kernel(x, edge_index, W, b):
    src = edge_index[0].astype(jnp.int32).reshape(NW, NCHUNK, K)
    dst = edge_index[1].astype(jnp.int32).reshape(NW, NCHUNK, K)
    ones16 = jnp.ones((K, HL), jnp.float32)
    z16 = jnp.zeros((N, HL), jnp.float32)
    z128 = jnp.zeros((N, C), jnp.float32)

    degp = _sc_degree(dst, ones16, z16)

    h = pl.pallas_call(
        _mm_body,
        grid=(N // BM,),
        in_specs=[
            pl.BlockSpec((BM, C), lambda i: (i, 0)),
            pl.BlockSpec((C, C), lambda i: (0, 0)),
        ],
        out_specs=pl.BlockSpec((BM, C), lambda i: (i, 0)),
        out_shape=jax.ShapeDtypeStruct((N, C), jnp.float32),
    )(x, W)

    h2 = pl.pallas_call(
        _prescale_body,
        grid=(N // BM,),
        in_specs=[
            pl.BlockSpec((NC, BM, HL), lambda i: (0, i, 0)),
            pl.BlockSpec((BM, C), lambda i: (i, 0)),
        ],
        out_specs=pl.BlockSpec((BM, C), lambda i: (i, 0)),
        out_shape=jax.ShapeDtypeStruct((N, C), jnp.float32),
    )(degp, h)

    acc = _sc_aggregate(src, dst, h2, z128)

    out = pl.pallas_call(
        _final_body,
        grid=(N // BM,),
        in_specs=[
            pl.BlockSpec((NC, BM, HL), lambda i: (0, i, 0)),
            pl.BlockSpec((NC, BM, C), lambda i: (0, i, 0)),
            pl.BlockSpec((BM, C), lambda i: (i, 0)),
            pl.BlockSpec((1, C), lambda i: (0, 0)),
        ],
        out_specs=pl.BlockSpec((BM, C), lambda i: (i, 0)),
        out_shape=jax.ShapeDtypeStruct((N, C), jnp.float32),
    )(degp, acc, h2, b.reshape(1, C))

    return out


# speedup vs baseline: 32.3597x; 32.3597x over previous
import dataclasses
import functools

import jax
import jax.numpy as jnp
from jax import lax
from jax.experimental import pallas as pl
from jax.experimental.pallas import tpu as pltpu
from jax.experimental.pallas import tpu_sc as plsc

N = 10000
E = 320000
C = 128
NC = 2
NS = 16
NW = NC * NS
EPW = E // NW
K = 80
NCHUNK = EPW // K

_mesh = plsc.VectorSubcoreMesh(core_axis_name="c", subcore_axis_name="s")

_cp_no_layout = pltpu.CompilerParams()
if "needs_layout_passes" in pltpu.CompilerParams.__dataclass_fields__:
    _cp_no_layout = dataclasses.replace(_cp_no_layout, needs_layout_passes=False)


@functools.partial(
    pl.kernel,
    out_type=jax.ShapeDtypeStruct((NW, N), jnp.float32),
    mesh=_mesh,
    scratch_types=[
        pltpu.VMEM((NCHUNK, K), jnp.int32),
        pltpu.VMEM((N,), jnp.float32),
    ],
    compiler_params=_cp_no_layout,
)
def _sc_degree(dst_hbm, degp_hbm, idx_v, deg_v):
    c = lax.axis_index("c")
    s = lax.axis_index("s")
    wid = c * NS + s
    pltpu.sync_copy(dst_hbm.at[wid], idx_v)

    zeros16 = jnp.zeros((16,), jnp.float32)

    @pl.loop(0, N // 16)
    def _(i):
        deg_v[pl.ds(i * 16, 16)] = zeros16

    ones16 = jnp.ones((16,), jnp.float32)

    @pl.loop(0, NCHUNK)
    def _(j):
        @pl.loop(0, K // 16)
        def _(l):
            idx16 = idx_v[j, pl.ds(l * 16, 16)]
            plsc.addupdate_scatter(deg_v, [idx16], ones16)

    pltpu.sync_copy(deg_v, degp_hbm.at[wid])


RPSA = 624
TBASE = RPSA * NS
TAIL = N - TBASE


def _rowwise_copy(s, src, dst):
    pltpu.sync_copy(src.at[pl.ds(s * RPSA, RPSA)], dst.at[pl.ds(s * RPSA, RPSA)])

    @pl.when(s == NS - 1)
    def _():
        pltpu.sync_copy(src.at[pl.ds(TBASE, TAIL)], dst.at[pl.ds(TBASE, TAIL)])


@functools.partial(
    pl.kernel,
    out_type=jax.ShapeDtypeStruct((NC, N, C), jnp.float32),
    mesh=_mesh,
    scratch_types=[
        pltpu.VMEM((2, K), jnp.int32),
        pltpu.VMEM((2, K), jnp.int32),
        pltpu.VMEM((K, C), jnp.float32),
        pltpu.VMEM((K, C), jnp.float32),
        pltpu.VMEM_SHARED((N, C), jnp.float32),
        pltpu.SemaphoreType.DMA,
        pltpu.SemaphoreType.DMA,
        pltpu.SemaphoreType.DMA,
        pltpu.SemaphoreType.DMA,
    ],
)
def _sc_aggregate(edges_hbm, z_hbm, h2_hbm, acc_hbm,
                  ia, ib, ra, rb, acc_sh, sia, sib, sa, sb):
    c = lax.axis_index("c")
    s = lax.axis_index("s")
    wid = c * NS + s
    _rowwise_copy(s, z_hbm, acc_sh)

    pltpu.sync_copy(edges_hbm.at[wid, 0], ia)
    plsc.subcore_barrier()
    pltpu.make_async_copy(h2_hbm.at[ia.at[0]], ra, sa).start()
    pltpu.make_async_copy(edges_hbm.at[wid, 1], ib, sib).start()

    @pl.loop(0, NCHUNK // 2)
    def _(j):
        c0 = 2 * j
        c1 = c0 + 1
        pltpu.make_async_copy(edges_hbm.at[wid, c1], ib, sib).wait()
        pltpu.make_async_copy(h2_hbm.at[ib.at[0]], rb, sb).start()
        pltpu.make_async_copy(h2_hbm.at[ia.at[0]], ra, sa).wait()
        pltpu.sync_copy(ra, acc_sh.at[ia.at[1]], add=True)
        pltpu.make_async_copy(edges_hbm.at[wid, c0 + 2], ia, sia).start()
        pltpu.make_async_copy(h2_hbm.at[ib.at[0]], rb, sb).wait()
        pltpu.sync_copy(rb, acc_sh.at[ib.at[1]], add=True)

        @pl.when(c1 + 2 < NCHUNK)
        def _():
            pltpu.make_async_copy(edges_hbm.at[wid, c1 + 2], ib, sib).start()

        pltpu.make_async_copy(edges_hbm.at[wid, c0 + 2], ia, sia).wait()
        pltpu.make_async_copy(h2_hbm.at[ia.at[0]], ra, sa).start()

    pltpu.make_async_copy(h2_hbm.at[ia.at[0]], ra, sa).wait()
    pltpu.sync_copy(ra, acc_sh.at[ia.at[1]], add=True)

    plsc.subcore_barrier()
    _rowwise_copy(s, acc_sh, acc_hbm.at[c])


BM = 1000


def _mm_body(x_ref, w_ref, h_ref):
    h_ref[...] = jnp.dot(x_ref[...], w_ref[...],
                         preferred_element_type=jnp.float32)


def _deg_block(degp_ref):
    return jnp.sum(degp_ref[...], axis=1, keepdims=True) + 1.0


def _prescale_body(degp_ref, h_ref, h2_ref):
    h2_ref[...] = lax.rsqrt(_deg_block(degp_ref)) * h_ref[...]


def _final_body(degp_ref, acc_ref, h2_ref, b_ref, o_ref):
    dis = lax.rsqrt(_deg_block(degp_ref))
    tot = acc_ref[0] + acc_ref[1] + h2_ref[...]
    o_ref[...] = jnp.maximum(dis * tot + b_ref[...], 0.0)


def kernel(x, edge_index, W, b):
    src = edge_index[0].astype(jnp.int32).reshape(NW, NCHUNK, K)
    dst = edge_index[1].astype(jnp.int32).reshape(NW, NCHUNK, K)
    edges = jnp.stack([src, dst], axis=2)
    zfull = jnp.zeros((N, C), jnp.float32)

    degp = _sc_degree(dst).T

    h = pl.pallas_call(
        _mm_body,
        grid=(N // BM,),
        in_specs=[
            pl.BlockSpec((BM, C), lambda i: (i, 0)),
            pl.BlockSpec((C, C), lambda i: (0, 0)),
        ],
        out_specs=pl.BlockSpec((BM, C), lambda i: (i, 0)),
        out_shape=jax.ShapeDtypeStruct((N, C), jnp.float32),
    )(x, W)

    h2 = pl.pallas_call(
        _prescale_body,
        grid=(N // BM,),
        in_specs=[
            pl.BlockSpec((BM, NW), lambda i: (i, 0)),
            pl.BlockSpec((BM, C), lambda i: (i, 0)),
        ],
        out_specs=pl.BlockSpec((BM, C), lambda i: (i, 0)),
        out_shape=jax.ShapeDtypeStruct((N, C), jnp.float32),
    )(degp, h)

    acc = _sc_aggregate(edges, zfull, h2)

    out = pl.pallas_call(
        _final_body,
        grid=(N // BM,),
        in_specs=[
            pl.BlockSpec((BM, NW), lambda i: (i, 0)),
            pl.BlockSpec((NC, BM, C), lambda i: (0, i, 0)),
            pl.BlockSpec((BM, C), lambda i: (i, 0)),
            pl.BlockSpec((1, C), lambda i: (0, 0)),
        ],
        out_specs=pl.BlockSpec((BM, C), lambda i: (i, 0)),
        out_shape=jax.ShapeDtypeStruct((N, C), jnp.float32),
    )(degp, acc, h2, b.reshape(1, C))

    return out


# speedup vs baseline: 33.7069x; 1.0416x over previous
import dataclasses
import functools

import jax
import jax.numpy as jnp
from jax import lax
from jax.experimental import pallas as pl
from jax.experimental.pallas import tpu as pltpu
from jax.experimental.pallas import tpu_sc as plsc

N = 10000
E = 320000
C = 128
NC = 2
NS = 16
NW = NC * NS
EPW = E // NW
K = 80
NCHUNK = EPW // K

_mesh = plsc.VectorSubcoreMesh(core_axis_name="c", subcore_axis_name="s")

_cp_no_layout = pltpu.CompilerParams()
if "needs_layout_passes" in pltpu.CompilerParams.__dataclass_fields__:
    _cp_no_layout = dataclasses.replace(_cp_no_layout, needs_layout_passes=False)


@functools.partial(
    pl.kernel,
    out_type=jax.ShapeDtypeStruct((NW, N), jnp.float32),
    mesh=_mesh,
    scratch_types=[
        pltpu.VMEM((NCHUNK, K), jnp.int32),
        pltpu.VMEM((N,), jnp.float32),
    ],
    compiler_params=_cp_no_layout,
)
def _sc_degree(dst_hbm, degp_hbm, idx_v, deg_v):
    c = lax.axis_index("c")
    s = lax.axis_index("s")
    wid = c * NS + s
    pltpu.sync_copy(dst_hbm.at[wid], idx_v)

    zeros16 = jnp.zeros((16,), jnp.float32)

    @pl.loop(0, N // 16)
    def _(i):
        deg_v[pl.ds(i * 16, 16)] = zeros16

    ones16 = jnp.ones((16,), jnp.float32)

    @pl.loop(0, NCHUNK)
    def _(j):
        @pl.loop(0, K // 16)
        def _(l):
            idx16 = idx_v[j, pl.ds(l * 16, 16)]
            plsc.addupdate_scatter(deg_v, [idx16], ones16)

    pltpu.sync_copy(deg_v, degp_hbm.at[wid])


RPSA = 624
TBASE = RPSA * NS
TAIL = N - TBASE


def _rowwise_copy(s, src, dst):
    pltpu.sync_copy(src.at[pl.ds(s * RPSA, RPSA)], dst.at[pl.ds(s * RPSA, RPSA)])

    @pl.when(s == NS - 1)
    def _():
        pltpu.sync_copy(src.at[pl.ds(TBASE, TAIL)], dst.at[pl.ds(TBASE, TAIL)])


@functools.partial(
    pl.kernel,
    out_type=jax.ShapeDtypeStruct((NC, N, C), jnp.float32),
    mesh=_mesh,
    scratch_types=[
        pltpu.VMEM((2, K), jnp.int32),
        pltpu.VMEM((2, K), jnp.int32),
        pltpu.VMEM((K, C), jnp.float32),
        pltpu.VMEM((K, C), jnp.float32),
        pltpu.VMEM_SHARED((N, C), jnp.float32),
        pltpu.SemaphoreType.DMA,
        pltpu.SemaphoreType.DMA,
        pltpu.SemaphoreType.DMA,
        pltpu.SemaphoreType.DMA,
    ],
)
def _sc_aggregate(edges_hbm, z_hbm, h2_hbm, acc_hbm,
                  ia, ib, ra, rb, acc_sh, sia, sib, sa, sb):
    c = lax.axis_index("c")
    s = lax.axis_index("s")
    wid = c * NS + s
    _rowwise_copy(s, z_hbm, acc_sh)

    pltpu.sync_copy(edges_hbm.at[wid, 0], ia)
    plsc.subcore_barrier()
    pltpu.make_async_copy(h2_hbm.at[ia.at[0]], ra, sa).start()
    pltpu.make_async_copy(edges_hbm.at[wid, 1], ib, sib).start()

    @pl.loop(0, NCHUNK // 2)
    def _(j):
        c0 = 2 * j
        c1 = c0 + 1
        pltpu.make_async_copy(edges_hbm.at[wid, c1], ib, sib).wait()
        pltpu.make_async_copy(h2_hbm.at[ib.at[0]], rb, sb).start()
        pltpu.make_async_copy(h2_hbm.at[ia.at[0]], ra, sa).wait()
        pltpu.sync_copy(ra, acc_sh.at[ia.at[1]], add=True)
        pltpu.make_async_copy(edges_hbm.at[wid, c0 + 2], ia, sia).start()
        pltpu.make_async_copy(h2_hbm.at[ib.at[0]], rb, sb).wait()
        pltpu.sync_copy(rb, acc_sh.at[ib.at[1]], add=True)

        @pl.when(c1 + 2 < NCHUNK)
        def _():
            pltpu.make_async_copy(edges_hbm.at[wid, c1 + 2], ib, sib).start()

        pltpu.make_async_copy(edges_hbm.at[wid, c0 + 2], ia, sia).wait()
        pltpu.make_async_copy(h2_hbm.at[ia.at[0]], ra, sa).start()

    pltpu.make_async_copy(h2_hbm.at[ia.at[0]], ra, sa).wait()
    pltpu.sync_copy(ra, acc_sh.at[ia.at[1]], add=True)

    plsc.subcore_barrier()
    _rowwise_copy(s, acc_sh, acc_hbm.at[c])


BM = 1000


def _deg_col(degp_ref):
    return jnp.sum(degp_ref[...], axis=1, keepdims=True) + 1.0


def _mmps_body(degp_ref, x_ref, w_ref, h2_ref):
    h = jnp.dot(x_ref[...], w_ref[...], preferred_element_type=jnp.float32)
    h2_ref[...] = lax.rsqrt(_deg_col(degp_ref)) * h


def _final_body(degp_ref, acc_ref, h2_ref, b_ref, o_ref):
    dis = lax.rsqrt(_deg_col(degp_ref))
    tot = acc_ref[0] + acc_ref[1] + h2_ref[...]
    o_ref[...] = jnp.maximum(dis * tot + b_ref[...], 0.0)


def kernel(x, edge_index, W, b):
    src = edge_index[0].astype(jnp.int32).reshape(NW, NCHUNK, K)
    dst = edge_index[1].astype(jnp.int32).reshape(NW, NCHUNK, K)
    edges = jnp.stack([src, dst], axis=2)
    zfull = jnp.zeros((N, C), jnp.float32)

    degp = _sc_degree(dst).T

    h2 = pl.pallas_call(
        _mmps_body,
        grid=(N // BM,),
        in_specs=[
            pl.BlockSpec((BM, NW), lambda i: (i, 0)),
            pl.BlockSpec((BM, C), lambda i: (i, 0)),
            pl.BlockSpec((C, C), lambda i: (0, 0)),
        ],
        out_specs=pl.BlockSpec((BM, C), lambda i: (i, 0)),
        out_shape=jax.ShapeDtypeStruct((N, C), jnp.float32),
    )(degp, x, W)

    acc = _sc_aggregate(edges, zfull, h2)

    out = pl.pallas_call(
        _final_body,
        grid=(N // BM,),
        in_specs=[
            pl.BlockSpec((BM, NW), lambda i: (i, 0)),
            pl.BlockSpec((NC, BM, C), lambda i: (0, i, 0)),
            pl.BlockSpec((BM, C), lambda i: (i, 0)),
            pl.BlockSpec((1, C), lambda i: (0, 0)),
        ],
        out_specs=pl.BlockSpec((BM, C), lambda i: (i, 0)),
        out_shape=jax.ShapeDtypeStruct((N, C), jnp.float32),
    )(degp, acc, h2, b.reshape(1, C))

    return out
